# Initial kernel scaffold; baseline (speedup 1.0000x reference)
import jax
import jax.numpy as jnp
from jax.experimental import pallas as pl


def kernel(batch, cluster_centers, cluster_counts):
    raise NotImplementedError("write your pallas kernel here")


# speedup vs baseline: 1.0453x; 1.0453x over previous
import functools

import jax
import jax.numpy as jnp
from jax import lax
from jax.experimental import pallas as pl
from jax.experimental.pallas import tpu as pltpu

K = 1024
D = 256
B = 8192
COLLAPSE_TOL = 0.5
_RB = 256
_NSTEP = B // _RB


def _assign_body(x_ref, ct_ref, out_ref):
    x = x_ref[...]
    ct = ct_ref[...]
    g = lax.dot_general(x, ct, (((1,), (0,)), ((), ())),
                        preferred_element_type=jnp.float32)
    b2 = jnp.sum(ct * ct, axis=0, keepdims=True)
    d2 = b2 - 2.0 * g
    out_ref[0, 0, :] = jnp.argmin(d2, axis=1).astype(jnp.int32)


def _assign(batch, centers_t):
    out = pl.pallas_call(
        _assign_body,
        grid=(_NSTEP,),
        in_specs=[
            pl.BlockSpec((_RB, D), lambda i: (i, 0)),
            pl.BlockSpec((D, K), lambda i: (0, 0)),
        ],
        out_specs=pl.BlockSpec((1, 1, _RB), lambda i: (i, 0, 0)),
        out_shape=jax.ShapeDtypeStruct((_NSTEP, 1, _RB), jnp.int32),
    )(batch, centers_t)
    return out.reshape(B)


def _update_body(c_ref, prev_ref, sums_ref, cb_ref, repl_ref,
                 c1_ref, lm_ref, lr_ref, nl_ref):
    f32 = jnp.float32
    c = c_ref[...]
    prev = prev_ref[...]
    sums = sums_ref[...]
    cb = cb_ref[...]
    empty = jnp.logical_and(prev == 0.0, cb == 0.0)
    cb = jnp.where(empty, 1.0, cb)
    sums = jnp.where(empty, repl_ref[...], sums)
    newc = prev + cb
    den = jnp.where(newc > 0.0, newc, 1.0)
    updated = (c * prev + sums) / den
    c1 = jnp.where(cb > 0.0, updated, c)
    c1_ref[...] = c1
    p = lax.dot_general(c1, c1, (((1,), (1,)), ((), ())),
                        preferred_element_type=f32)
    n2 = jnp.sum(c1 * c1, axis=1, keepdims=True)
    ones = jnp.ones((K, 1), f32)
    n2t = lax.dot_general(ones, n2, (((1,), (1,)), ((), ())),
                          preferred_element_type=f32)
    d2p = n2 + n2t - 2.0 * p
    rowi = lax.broadcasted_iota(jnp.int32, (K, K), 0)
    colj = lax.broadcasted_iota(jnp.int32, (K, K), 1)
    close = jnp.logical_and(d2p < COLLAPSE_TOL * COLLAPSE_TOL, colj > rowi)
    newct = lax.dot_general(ones, newc, (((1,), (1,)), ((), ())),
                            preferred_element_type=f32)
    cnt_le = newc <= newct
    li = jnp.max(jnp.where(jnp.logical_and(close, cnt_le), 1.0, 0.0),
                 axis=1, keepdims=True)
    ljsrc = jnp.where(jnp.logical_and(close, jnp.logical_not(cnt_le)), 1.0, 0.0)
    ljc = lax.dot_general(ljsrc, ones, (((0,), (0,)), ((), ())),
                          preferred_element_type=f32)
    lm = jnp.maximum(li, jnp.where(ljc > 0.0, 1.0, 0.0))
    lm_ref[...] = lm
    tril = (colj <= rowi).astype(f32)
    rank = lax.dot_general(tril, lm, (((1,), (0,)), ((), ())),
                           preferred_element_type=f32)
    nl_ref[...] = jnp.sum(lm, axis=(0, 1), keepdims=True)
    lr_ref[...] = jnp.clip(rank - 1.0, 0.0, float(B - 1)).astype(jnp.int32)


def _update(centers, prev_counts, sums, counts_batch, repl):
    return pl.pallas_call(
        _update_body,
        out_shape=[
            jax.ShapeDtypeStruct((K, D), jnp.float32),
            jax.ShapeDtypeStruct((K, 1), jnp.float32),
            jax.ShapeDtypeStruct((K, 1), jnp.int32),
            jax.ShapeDtypeStruct((1, 1), jnp.float32),
        ],
    )(centers, prev_counts, sums, counts_batch, repl)


def _far_body(x_ref, ct1_ref, out_ref):
    x = x_ref[...]
    ct1 = ct1_ref[...]
    g = lax.dot_general(x, ct1, (((1,), (0,)), ((), ())),
                        preferred_element_type=jnp.float32)
    a2 = jnp.sum(x * x, axis=1, keepdims=True)
    b2 = jnp.sum(ct1 * ct1, axis=0, keepdims=True)
    d2 = jnp.maximum(a2 + b2 - 2.0 * g, 0.0)
    out_ref[0, 0, :] = jnp.max(d2, axis=1)


def _farthest(batch, centers1_t):
    out = pl.pallas_call(
        _far_body,
        grid=(_NSTEP,),
        in_specs=[
            pl.BlockSpec((_RB, D), lambda i: (i, 0)),
            pl.BlockSpec((D, K), lambda i: (0, 0)),
        ],
        out_specs=pl.BlockSpec((1, 1, _RB), lambda i: (i, 0, 0)),
        out_shape=jax.ShapeDtypeStruct((_NSTEP, 1, _RB), jnp.float32),
    )(batch, centers1_t)
    return out.reshape(B)


def _scatter(batch, assignments):
    counts = jnp.zeros((K,), jnp.float32).at[assignments].add(1.0)
    sums = jnp.zeros((K, D), jnp.float32).at[assignments].add(batch)
    return sums, counts


def kernel(batch, cluster_centers, cluster_counts):
    assignments = _assign(batch, cluster_centers.T)
    sums, counts_batch = _scatter(batch, assignments)
    repl_idx = jax.random.randint(jax.random.key(1), (K,), 0, B)
    replacement = batch[repl_idx]
    centers1, lm, lr, nl = _update(
        cluster_centers, cluster_counts.reshape(K, 1), sums,
        counts_batch.reshape(K, 1), replacement)

    def split_branch():
        far = _farthest(batch, centers1.T)
        _, order = lax.top_k(far, K)
        repl2 = batch[order[lr.reshape(K)]]
        return jnp.where(lm > 0.0, repl2, centers1)

    return lax.cond(nl[0, 0] > 0.0, split_branch, lambda: centers1)


# speedup vs baseline: 2.1013x; 2.0101x over previous
import functools

import jax
import jax.numpy as jnp
from jax import lax
from jax.experimental import pallas as pl
from jax.experimental.pallas import tpu as pltpu
from jax.experimental.pallas import tpu_sc as plsc

K = 1024
D = 256
B = 8192
COLLAPSE_TOL = 0.5
_RB = 256
_NSTEP = B // _RB


def _assign_body(x_ref, ct_ref, out_ref, cnt_ref):
    x = x_ref[...]
    ct = ct_ref[...]
    g = lax.dot_general(x, ct, (((1,), (0,)), ((), ())),
                        preferred_element_type=jnp.float32)
    b2 = jnp.sum(ct * ct, axis=0, keepdims=True)
    d2 = b2 - 2.0 * g
    out_ref[0, 0, :] = jnp.argmin(d2, axis=1).astype(jnp.int32)
    m = jnp.min(d2, axis=1, keepdims=True)
    oh = (d2 == m).astype(jnp.float32)
    cnt_ref[0, 0, :] = jnp.sum(oh, axis=0)


def _assign(batch, centers_t):
    out, cnt = pl.pallas_call(
        _assign_body,
        grid=(_NSTEP,),
        in_specs=[
            pl.BlockSpec((_RB, D), lambda i: (i, 0)),
            pl.BlockSpec((D, K), lambda i: (0, 0)),
        ],
        out_specs=[
            pl.BlockSpec((1, 1, _RB), lambda i: (i, 0, 0)),
            pl.BlockSpec((1, 1, K), lambda i: (i, 0, 0)),
        ],
        out_shape=[
            jax.ShapeDtypeStruct((_NSTEP, 1, _RB), jnp.int32),
            jax.ShapeDtypeStruct((_NSTEP, 1, K), jnp.float32),
        ],
    )(batch, centers_t)
    return out.reshape(B), cnt.reshape(_NSTEP, K)


_NC = 2
_NS = 16
_CHUNK = B // (_NC * _NS)
_RW = K // _NS


def _sc_body(batch_hbm, asg_hbm, ridx_hbm, zrow_hbm,
             sums_hbm, repl_hbm,
             rows_v, idx_v, ridx_v, rbuf_v, tab_sh, sem):
    c = lax.axis_index("c")
    s = lax.axis_index("s")
    g = c * _NS + s
    pltpu.sync_copy(zrow_hbm, tab_sh.at[pl.ds(s * _RW, _RW)])
    pltpu.sync_copy(batch_hbm.at[pl.ds(g * _CHUNK, _CHUNK)], rows_v)
    pltpu.sync_copy(asg_hbm.at[pl.ds(g * 2, 2)], idx_v)
    pltpu.sync_copy(ridx_hbm.at[pl.ds(g * 32, 32)], ridx_v)
    pltpu.async_copy(batch_hbm.at[ridx_v], rbuf_v, sem).wait()
    pltpu.sync_copy(rbuf_v, repl_hbm.at[g])
    plsc.subcore_barrier()
    for j in range(2):
        pltpu.sync_copy(rows_v.at[pl.ds(j * 128, 128)],
                        tab_sh.at[idx_v.at[j]], add=True)
    plsc.subcore_barrier()
    pltpu.sync_copy(tab_sh.at[pl.ds(s * _RW, _RW)],
                    sums_hbm.at[c, pl.ds(s * _RW, _RW)])


def _sc_scatter(batch, assignments, repl_idx):
    mesh = plsc.VectorSubcoreMesh(core_axis_name="c", subcore_axis_name="s")
    f = pl.kernel(
        _sc_body,
        out_type=[
            jax.ShapeDtypeStruct((_NC, K, D), jnp.float32),
            jax.ShapeDtypeStruct((_NC * _NS, 32, D), jnp.float32),
        ],
        mesh=mesh,
        scratch_types=[
            pltpu.VMEM((_CHUNK, D), jnp.float32),
            pltpu.VMEM((2, 128), jnp.int32),
            pltpu.VMEM((32,), jnp.int32),
            pltpu.VMEM((32, D), jnp.float32),
            pltpu.VMEM_SHARED((K, D), jnp.float32),
            pltpu.SemaphoreType.DMA,
        ],
        compiler_params=pltpu.CompilerParams(use_tc_tiling_on_sc=False),
    )
    sums, repl = f(batch, assignments.reshape(64, 128), repl_idx,
                   jnp.zeros((_RW, D), jnp.float32))
    return sums[0] + sums[1], repl.reshape(K, D)


def _update_body(c_ref, prev_ref, sums_ref, cp_ref, repl_ref,
                 c1_ref, lm_ref, lr_ref, nl_ref):
    f32 = jnp.float32
    c = c_ref[...]
    prev = prev_ref[...]
    sums = sums_ref[...]
    ones_n = jnp.ones((_NSTEP, 1), f32)
    cb = lax.dot_general(cp_ref[...], ones_n, (((0,), (0,)), ((), ())),
                         preferred_element_type=f32)
    empty = jnp.logical_and(prev == 0.0, cb == 0.0)
    cb = jnp.where(empty, 1.0, cb)
    sums = jnp.where(empty, repl_ref[...], sums)
    newc = prev + cb
    den = jnp.where(newc > 0.0, newc, 1.0)
    updated = (c * prev + sums) / den
    c1 = jnp.where(cb > 0.0, updated, c)
    c1_ref[...] = c1
    p = lax.dot_general(c1, c1, (((1,), (1,)), ((), ())),
                        preferred_element_type=f32)
    n2 = jnp.sum(c1 * c1, axis=1, keepdims=True)
    ones = jnp.ones((K, 1), f32)
    n2t = lax.dot_general(ones, n2, (((1,), (1,)), ((), ())),
                          preferred_element_type=f32)
    d2p = n2 + n2t - 2.0 * p
    rowi = lax.broadcasted_iota(jnp.int32, (K, K), 0)
    colj = lax.broadcasted_iota(jnp.int32, (K, K), 1)
    close = jnp.logical_and(d2p < COLLAPSE_TOL * COLLAPSE_TOL, colj > rowi)
    newct = lax.dot_general(ones, newc, (((1,), (1,)), ((), ())),
                            preferred_element_type=f32)
    cnt_le = newc <= newct
    li = jnp.max(jnp.where(jnp.logical_and(close, cnt_le), 1.0, 0.0),
                 axis=1, keepdims=True)
    ljsrc = jnp.where(jnp.logical_and(close, jnp.logical_not(cnt_le)), 1.0, 0.0)
    ljc = lax.dot_general(ljsrc, ones, (((0,), (0,)), ((), ())),
                          preferred_element_type=f32)
    lm = jnp.maximum(li, jnp.where(ljc > 0.0, 1.0, 0.0))
    lm_ref[...] = lm
    tril = (colj <= rowi).astype(f32)
    rank = lax.dot_general(tril, lm, (((1,), (0,)), ((), ())),
                           preferred_element_type=f32)
    nl_ref[...] = jnp.sum(lm, axis=(0, 1), keepdims=True)
    lr_ref[...] = jnp.clip(rank - 1.0, 0.0, float(B - 1)).astype(jnp.int32)


def _update(centers, prev_counts, sums, cnt_parts, repl):
    return pl.pallas_call(
        _update_body,
        out_shape=[
            jax.ShapeDtypeStruct((K, D), jnp.float32),
            jax.ShapeDtypeStruct((K, 1), jnp.float32),
            jax.ShapeDtypeStruct((K, 1), jnp.int32),
            jax.ShapeDtypeStruct((1, 1), jnp.float32),
        ],
    )(centers, prev_counts, sums, cnt_parts, repl)


def _far_body(x_ref, ct1_ref, out_ref):
    x = x_ref[...]
    ct1 = ct1_ref[...]
    g = lax.dot_general(x, ct1, (((1,), (0,)), ((), ())),
                        preferred_element_type=jnp.float32)
    a2 = jnp.sum(x * x, axis=1, keepdims=True)
    b2 = jnp.sum(ct1 * ct1, axis=0, keepdims=True)
    d2 = jnp.maximum(a2 + b2 - 2.0 * g, 0.0)
    out_ref[0, 0, :] = jnp.max(d2, axis=1)


def _farthest(batch, centers1_t):
    out = pl.pallas_call(
        _far_body,
        grid=(_NSTEP,),
        in_specs=[
            pl.BlockSpec((_RB, D), lambda i: (i, 0)),
            pl.BlockSpec((D, K), lambda i: (0, 0)),
        ],
        out_specs=pl.BlockSpec((1, 1, _RB), lambda i: (i, 0, 0)),
        out_shape=jax.ShapeDtypeStruct((_NSTEP, 1, _RB), jnp.float32),
    )(batch, centers1_t)
    return out.reshape(B)


def kernel(batch, cluster_centers, cluster_counts):
    assignments, cnt_parts = _assign(batch, cluster_centers.T)
    repl_idx = jax.random.randint(jax.random.key(1), (K,), 0, B)
    sums, replacement = _sc_scatter(batch, assignments, repl_idx)
    centers1, lm, lr, nl = _update(
        cluster_centers, cluster_counts.reshape(K, 1), sums,
        cnt_parts, replacement)

    def split_branch():
        far = _farthest(batch, centers1.T)
        _, order = lax.top_k(far, K)
        repl2 = batch[order[lr.reshape(K)]]
        return jnp.where(lm > 0.0, repl2, centers1)

    return lax.cond(nl[0, 0] > 0.0, split_branch, lambda: centers1)


# speedup vs baseline: 2.1947x; 1.0444x over previous
import functools

import jax
import jax.numpy as jnp
from jax import lax
from jax.experimental import pallas as pl
from jax.experimental.pallas import tpu as pltpu
from jax.experimental.pallas import tpu_sc as plsc

K = 1024
D = 256
B = 8192
COLLAPSE_TOL = 0.5
_RB = 256
_NSTEP = B // _RB


def _assign_body(x_ref, ct_ref, out_ref, cnt_ref):
    x = x_ref[...]
    ct = ct_ref[...]
    g = lax.dot_general(x, ct, (((1,), (0,)), ((), ())),
                        preferred_element_type=jnp.float32)
    b2 = jnp.sum(ct * ct, axis=0, keepdims=True)
    d2 = b2 - 2.0 * g
    out_ref[0, 0, :] = jnp.argmin(d2, axis=1).astype(jnp.int32)
    m = jnp.min(d2, axis=1, keepdims=True)
    oh = (d2 == m).astype(jnp.float32)
    cnt_ref[0, 0, :] = jnp.sum(oh, axis=0)


def _assign(batch, centers_t):
    out, cnt = pl.pallas_call(
        _assign_body,
        grid=(_NSTEP,),
        in_specs=[
            pl.BlockSpec((_RB, D), lambda i: (i, 0)),
            pl.BlockSpec((D, K), lambda i: (0, 0)),
        ],
        out_specs=[
            pl.BlockSpec((1, 1, _RB), lambda i: (i, 0, 0)),
            pl.BlockSpec((1, 1, K), lambda i: (i, 0, 0)),
        ],
        out_shape=[
            jax.ShapeDtypeStruct((_NSTEP, 1, _RB), jnp.int32),
            jax.ShapeDtypeStruct((_NSTEP, 1, K), jnp.float32),
        ],
    )(batch, centers_t)
    return out.reshape(B), cnt.reshape(_NSTEP, K)


_NC = 2
_NS = 16
_CHUNK = B // (_NC * _NS)
_RW = K // _NS


def _sc_body(batch_hbm, asg_hbm, ridx_hbm, zrow_hbm,
             sums_hbm, repl_hbm,
             rows_v, idx_v, ridx_v, rbuf_v, tab_sh,
             sem0, sem1, sem2, sem3, sem4, sem5):
    c = lax.axis_index("c")
    s = lax.axis_index("s")
    g = c * _NS + s
    zcp = pltpu.async_copy(zrow_hbm, tab_sh.at[pl.ds(s * _RW, _RW)], sem0)
    rcp = pltpu.async_copy(batch_hbm.at[pl.ds(g * _CHUNK, _CHUNK)], rows_v,
                           sem1)
    icp = pltpu.async_copy(asg_hbm.at[pl.ds(g * 2, 2)], idx_v, sem2)
    xcp = pltpu.async_copy(ridx_hbm.at[pl.ds(g * 32, 32)], ridx_v, sem3)
    xcp.wait()
    gcp = pltpu.async_copy(batch_hbm.at[ridx_v], rbuf_v, sem3)
    zcp.wait()
    rcp.wait()
    icp.wait()
    plsc.subcore_barrier()
    s0 = pltpu.async_copy(rows_v.at[pl.ds(0, 128)],
                          tab_sh.at[idx_v.at[0]], sem4, add=True)
    s1 = pltpu.async_copy(rows_v.at[pl.ds(128, 128)],
                          tab_sh.at[idx_v.at[1]], sem5, add=True)
    gcp.wait()
    wcp = pltpu.async_copy(rbuf_v, repl_hbm.at[g], sem3)
    s0.wait()
    s1.wait()
    wcp.wait()
    plsc.subcore_barrier()
    pltpu.sync_copy(tab_sh.at[pl.ds(s * _RW, _RW)],
                    sums_hbm.at[c, pl.ds(s * _RW, _RW)])


def _sc_scatter(batch, assignments, repl_idx):
    mesh = plsc.VectorSubcoreMesh(core_axis_name="c", subcore_axis_name="s")
    f = pl.kernel(
        _sc_body,
        out_type=[
            jax.ShapeDtypeStruct((_NC, K, D), jnp.float32),
            jax.ShapeDtypeStruct((_NC * _NS, 32, D), jnp.float32),
        ],
        mesh=mesh,
        scratch_types=[
            pltpu.VMEM((_CHUNK, D), jnp.float32),
            pltpu.VMEM((2, 128), jnp.int32),
            pltpu.VMEM((32,), jnp.int32),
            pltpu.VMEM((32, D), jnp.float32),
            pltpu.VMEM_SHARED((K, D), jnp.float32),
            pltpu.SemaphoreType.DMA,
            pltpu.SemaphoreType.DMA,
            pltpu.SemaphoreType.DMA,
            pltpu.SemaphoreType.DMA,
            pltpu.SemaphoreType.DMA,
            pltpu.SemaphoreType.DMA,
        ],
        compiler_params=pltpu.CompilerParams(use_tc_tiling_on_sc=False),
    )
    sums, repl = f(batch, assignments.reshape(64, 128), repl_idx,
                   jnp.zeros((_RW, D), jnp.float32))
    return sums[0] + sums[1], repl.reshape(K, D)


def _update_body(c_ref, prev_ref, sums_ref, cp_ref, repl_ref,
                 c1_ref, lm_ref, lr_ref, nl_ref):
    f32 = jnp.float32
    c = c_ref[...]
    prev = prev_ref[...]
    sums = sums_ref[...]
    ones_n = jnp.ones((_NSTEP, 1), f32)
    cb = lax.dot_general(cp_ref[...], ones_n, (((0,), (0,)), ((), ())),
                         preferred_element_type=f32)
    empty = jnp.logical_and(prev == 0.0, cb == 0.0)
    cb = jnp.where(empty, 1.0, cb)
    sums = jnp.where(empty, repl_ref[...], sums)
    newc = prev + cb
    den = jnp.where(newc > 0.0, newc, 1.0)
    updated = (c * prev + sums) / den
    c1 = jnp.where(cb > 0.0, updated, c)
    c1_ref[...] = c1
    p = lax.dot_general(c1, c1, (((1,), (1,)), ((), ())),
                        preferred_element_type=f32)
    n2 = jnp.sum(c1 * c1, axis=1, keepdims=True)
    ones = jnp.ones((K, 1), f32)
    n2t = lax.dot_general(ones, n2, (((1,), (1,)), ((), ())),
                          preferred_element_type=f32)
    d2p = n2 + n2t - 2.0 * p
    rowi = lax.broadcasted_iota(jnp.int32, (K, K), 0)
    colj = lax.broadcasted_iota(jnp.int32, (K, K), 1)
    close = jnp.logical_and(d2p < COLLAPSE_TOL * COLLAPSE_TOL, colj > rowi)
    newct = lax.dot_general(ones, newc, (((1,), (1,)), ((), ())),
                            preferred_element_type=f32)
    cnt_le = newc <= newct
    li = jnp.max(jnp.where(jnp.logical_and(close, cnt_le), 1.0, 0.0),
                 axis=1, keepdims=True)
    ljsrc = jnp.where(jnp.logical_and(close, jnp.logical_not(cnt_le)), 1.0, 0.0)
    ljc = lax.dot_general(ljsrc, ones, (((0,), (0,)), ((), ())),
                          preferred_element_type=f32)
    lm = jnp.maximum(li, jnp.where(ljc > 0.0, 1.0, 0.0))
    lm_ref[...] = lm
    tril = (colj <= rowi).astype(f32)
    rank = lax.dot_general(tril, lm, (((1,), (0,)), ((), ())),
                           preferred_element_type=f32)
    nl_ref[...] = jnp.sum(lm, axis=(0, 1), keepdims=True)
    lr_ref[...] = jnp.clip(rank - 1.0, 0.0, float(B - 1)).astype(jnp.int32)


def _update(centers, prev_counts, sums, cnt_parts, repl):
    return pl.pallas_call(
        _update_body,
        out_shape=[
            jax.ShapeDtypeStruct((K, D), jnp.float32),
            jax.ShapeDtypeStruct((K, 1), jnp.float32),
            jax.ShapeDtypeStruct((K, 1), jnp.int32),
            jax.ShapeDtypeStruct((1, 1), jnp.float32),
        ],
    )(centers, prev_counts, sums, cnt_parts, repl)


def _far_body(x_ref, ct1_ref, out_ref):
    x = x_ref[...]
    ct1 = ct1_ref[...]
    g = lax.dot_general(x, ct1, (((1,), (0,)), ((), ())),
                        preferred_element_type=jnp.float32)
    a2 = jnp.sum(x * x, axis=1, keepdims=True)
    b2 = jnp.sum(ct1 * ct1, axis=0, keepdims=True)
    d2 = jnp.maximum(a2 + b2 - 2.0 * g, 0.0)
    out_ref[0, 0, :] = jnp.max(d2, axis=1)


def _farthest(batch, centers1_t):
    out = pl.pallas_call(
        _far_body,
        grid=(_NSTEP,),
        in_specs=[
            pl.BlockSpec((_RB, D), lambda i: (i, 0)),
            pl.BlockSpec((D, K), lambda i: (0, 0)),
        ],
        out_specs=pl.BlockSpec((1, 1, _RB), lambda i: (i, 0, 0)),
        out_shape=jax.ShapeDtypeStruct((_NSTEP, 1, _RB), jnp.float32),
    )(batch, centers1_t)
    return out.reshape(B)


def kernel(batch, cluster_centers, cluster_counts):
    assignments, cnt_parts = _assign(batch, cluster_centers.T)
    repl_idx = jax.random.randint(jax.random.key(1), (K,), 0, B)
    sums, replacement = _sc_scatter(batch, assignments, repl_idx)
    centers1, lm, lr, nl = _update(
        cluster_centers, cluster_counts.reshape(K, 1), sums,
        cnt_parts, replacement)

    def split_branch():
        far = _farthest(batch, centers1.T)
        _, order = lax.top_k(far, K)
        repl2 = batch[order[lr.reshape(K)]]
        return jnp.where(lm > 0.0, repl2, centers1)

    return lax.cond(nl[0, 0] > 0.0, split_branch, lambda: centers1)
